# baseline (speedup 1.0000x reference)
import functools

import jax
import jax.numpy as jnp
from jax import lax
from jax.experimental import pallas as pl
from jax.experimental.pallas import tpu as pltpu
from jax.experimental.pallas import tpu_sc as plsc

N = 50000
E = 800000
D_IN = 128
DH = 64
NG = 256
HALF = DH // 2

NP = 50176
NDUM = NP - N
NPOOL = 1280

C = 448
KM = 110
E_MAIN = 16 * KM * C
KT = 10
TAIL = 16 * KT * C
KP = 8
EPP = 16 * KP * C

BN = 1000


def _make_edge_scatter(num_rows, mode):
  mesh = plsc.VectorSubcoreMesh(core_axis_name="c", subcore_axis_name="s")
  zr = num_rows // 16
  gather = mode != "deg"

  scratch = []
  if gather:
    scratch += [pltpu.VMEM((C,), jnp.int32)] * 2
  scratch += [pltpu.VMEM((C,), jnp.int32)] * 2
  scratch += [pltpu.VMEM((C, HALF), jnp.float32)] * (2 if gather else 1)
  scratch += [
      pltpu.VMEM_SHARED((num_rows, HALF), jnp.float32),
  ]
  scratch += [pltpu.SemaphoreType.DMA] * 4

  def body(*refs):
    if mode == "conv":
      (table, edge, tsrc, tdst, out, sixa, sixb, dixa, dixb, rwa, rwb, acc,
       sga, sgb, ssa, ssb) = refs
    elif mode == "pool":
      (table, src1, dst1, out, sixa, sixb, dixa, dixb, rwa, rwb, acc,
       sga, sgb, ssa, ssb) = refs
    else:
      edge, tdst, out, dixa, dixb, rwa, acc, sga, sgb, ssa, ssb = refs
      rwb = rwa
    c = lax.axis_index("c")
    s = lax.axis_index("s")

    def fill_rows(val):
      v16 = jnp.full((16,), val, jnp.float32)
      def fb(r, carry):
        for j in range(HALF // 16):
          rwa[r, pl.ds(j * 16, 16)] = v16
        return carry
      lax.fori_loop(0, C, fb, 0)

    fill_rows(0.0)
    row0 = s * zr
    off = 0
    while off < zr:
      step = min(C, zr - off)
      pltpu.sync_copy(rwa.at[pl.ds(0, step)], acc.at[pl.ds(row0 + off, step)])
      off += step
    plsc.subcore_barrier()

    if not gather:
      fill_rows(1.0)

    if gather:
      tbl = table.at[c]

    def gat(six, rw, sem):
      return pltpu.make_async_copy(tbl.at[six], rw, sem)

    def sca(rw, dix, sem):
      return pltpu.make_async_copy(rw, acc.at[dix], sem)

    def load_main(k, six, dix):
      base = s * (KM * C) + k * C
      pltpu.sync_copy(edge.at[1, pl.ds(base, C)], dix)
      if gather:
        pltpu.sync_copy(edge.at[0, pl.ds(base, C)], six)

    def load_tail(k, six, dix):
      base = s * (KT * C) + k * C
      pltpu.sync_copy(tdst.at[pl.ds(base, C)], dix)
      if gather:
        pltpu.sync_copy(tsrc.at[pl.ds(base, C)], six)

    def load_pool(k, six, dix):
      base = s * (KP * C) + k * C
      pltpu.sync_copy(dst1.at[pl.ds(base, C)], dix)
      pltpu.sync_copy(src1.at[pl.ds(base, C)], six)

    def pipe(k_chunks, load):
      load(0, sixa, dixa)
      gat(sixa, rwa, sga).start()

      def pair(m, carry):
        load(2 * m + 1, sixb, dixb)
        gat(sixb, rwb, sgb).start()
        gat(sixa, rwa, sga).wait()
        sca(rwa, dixa, ssa).start(add=True)
        gat(sixb, rwb, sgb).wait()
        sca(rwb, dixb, ssb).start(add=True)
        sca(rwa, dixa, ssa).wait()
        load(2 * m + 2, sixa, dixa)
        gat(sixa, rwa, sga).start()
        sca(rwb, dixb, ssb).wait()
        return carry

      lax.fori_loop(0, k_chunks // 2 - 1, pair, 0)
      load(k_chunks - 1, sixb, dixb)
      gat(sixb, rwb, sgb).start()
      gat(sixa, rwa, sga).wait()
      sca(rwa, dixa, ssa).start(add=True)
      gat(sixb, rwb, sgb).wait()
      sca(rwb, dixb, ssb).start(add=True)
      sca(rwa, dixa, ssa).wait()
      sca(rwb, dixb, ssb).wait()

    def pipe_scatter_only(k_chunks, load):
      load(0, None, dixa)

      def pair(m, carry):
        sca(rwa, dixa, ssa).start(add=True)
        load(2 * m + 1, None, dixb)
        sca(rwb, dixb, ssb).start(add=True)
        sca(rwa, dixa, ssa).wait()
        load(2 * m + 2, None, dixa)
        sca(rwb, dixb, ssb).wait()
        return carry

      lax.fori_loop(0, k_chunks // 2 - 1, pair, 0)
      sca(rwa, dixa, ssa).start(add=True)
      load(k_chunks - 1, None, dixb)
      sca(rwb, dixb, ssb).start(add=True)
      sca(rwa, dixa, ssa).wait()
      sca(rwb, dixb, ssb).wait()

    if mode == "conv":
      pipe(KM, load_main)
      pipe(KT, load_tail)
    elif mode == "pool":
      pipe(KP, load_pool)
    else:
      pipe_scatter_only(KM, load_main)
      pipe_scatter_only(KT, load_tail)

    plsc.subcore_barrier()

    off = 0
    while off < zr:
      step = min(C, zr - off)
      pltpu.sync_copy(acc.at[pl.ds(row0 + off, step)], rwa.at[pl.ds(0, step)])
      pltpu.sync_copy(rwa.at[pl.ds(0, step)],
                      out.at[c, pl.ds(row0 + off, step)])
      off += step

  return pl.kernel(
      body,
      out_type=jax.ShapeDtypeStruct((2, num_rows, HALF), jnp.float32),
      mesh=mesh,
      scratch_types=scratch,
      compiler_params=pltpu.CompilerParams(use_tc_tiling_on_sc=False),
  )


_deg_sc = _make_edge_scatter(NP, "deg")
_conv_sc = _make_edge_scatter(NP, "conv")
_pool_sc = _make_edge_scatter(NPOOL, "pool")


def _split_store(tab_ref, t):
  tab_ref[0] = t[:, :HALF]
  tab_ref[1] = t[:, HALF:]


def _tc0_body(x_ref, deg_ref, wlin_ref, blin_ref, w1_ref, tab_ref, dis_ref):
  dis = lax.rsqrt(deg_ref[0][:, :1])
  dis_ref[...] = dis
  h0 = jnp.dot(x_ref[...], wlin_ref[...], preferred_element_type=jnp.float32)
  h0 = h0 + blin_ref[...]
  t1 = jnp.dot(h0, w1_ref[...], preferred_element_type=jnp.float32)
  _split_store(tab_ref, t1 * dis)


def _tc_mid_body(acc_ref, dis_ref, b_ref, w_ref, tab_ref):
  dis = dis_ref[...]
  m = jnp.concatenate([acc_ref[0], acc_ref[1]], axis=1)
  conv = jnp.maximum(dis * m + b_ref[...], 0.0)
  t = jnp.dot(conv, w_ref[...], preferred_element_type=jnp.float32)
  _split_store(tab_ref, t * dis)


def _tc_emb_body(acc_ref, dis_ref, b_ref, wemb_ref, bemb_ref, tab_ref):
  m = jnp.concatenate([acc_ref[0], acc_ref[1]], axis=1)
  conv = jnp.maximum(dis_ref[...] * m + b_ref[...], 0.0)
  he = jnp.dot(conv, wemb_ref[...], preferred_element_type=jnp.float32)
  he = jnp.maximum(he + bemb_ref[...], 0.0)
  _split_store(tab_ref, he)


def _tc_pred_body(accp_ref, wpred_ref, bpred_ref, out_ref):
  g = jnp.concatenate([accp_ref[0], accp_ref[1]], axis=1)
  out_ref[...] = (
      jnp.dot(g, wpred_ref[...], preferred_element_type=jnp.float32)
      + bpred_ref[...]
  )


def _full(shape):
  return pl.BlockSpec(shape, lambda i: tuple(0 for _ in shape))

_GRID = N // BN
_acc_spec = pl.BlockSpec((2, BN, HALF), lambda i: (0, i, 0))
_dis_spec = pl.BlockSpec((BN, 1), lambda i: (i, 0))
_tab_spec = pl.BlockSpec((2, BN, HALF), lambda i: (0, i, 0))

_tab_shape = jax.ShapeDtypeStruct((2, N, HALF), jnp.float32)
_dis_shape = jax.ShapeDtypeStruct((N, 1), jnp.float32)

_tc0 = pl.pallas_call(
    _tc0_body,
    grid=(_GRID,),
    in_specs=[
        pl.BlockSpec((BN, D_IN), lambda i: (i, 0)),
        _acc_spec,
        _full((D_IN, DH)),
        _full((1, DH)),
        _full((DH, DH)),
    ],
    out_specs=[_tab_spec, _dis_spec],
    out_shape=[_tab_shape, _dis_shape],
)

_tc_mid = pl.pallas_call(
    _tc_mid_body,
    grid=(_GRID,),
    in_specs=[_acc_spec, _dis_spec, _full((1, DH)), _full((DH, DH))],
    out_specs=[_tab_spec],
    out_shape=[_tab_shape],
)

_tc_emb = pl.pallas_call(
    _tc_emb_body,
    grid=(_GRID,),
    in_specs=[_acc_spec, _dis_spec, _full((1, DH)), _full((DH, DH)),
              _full((1, DH))],
    out_specs=[_tab_spec],
    out_shape=[_tab_shape],
)

_tc_pred = pl.pallas_call(
    _tc_pred_body,
    grid=(1,),
    in_specs=[
        pl.BlockSpec((2, NG, HALF), lambda i: (0, 0, 0)),
        _full((DH, 1)),
        _full((1, 1)),
    ],
    out_specs=[pl.BlockSpec((NG, 1), lambda i: (0, 0))],
    out_shape=[jax.ShapeDtypeStruct((NG, 1), jnp.float32)],
)


def kernel(x, edge_index, batch, W_lin, b_lin, W1, b1, W2, b2, W3, b3,
           W_emb, b_emb, W_pred, b_pred):
  i32 = jnp.int32
  iota_n = jnp.arange(N, dtype=i32)

  npad = TAIL - (E - E_MAIN) - N
  pad = jnp.arange(npad, dtype=i32)
  tail_src = jnp.concatenate([edge_index[0, E_MAIN:], iota_n, pad % N])
  tail_dst = jnp.concatenate([edge_index[1, E_MAIN:], iota_n,
                              N + pad % NDUM])

  ppad = jnp.arange(EPP - N, dtype=i32)
  src_p = jnp.concatenate([iota_n, ppad % N])
  dst_p = jnp.concatenate([batch, NG + ppad % (NPOOL - NG)])

  b_lin2 = b_lin.reshape(1, DH)
  b1_2, b2_2, b3_2 = b1.reshape(1, DH), b2.reshape(1, DH), b3.reshape(1, DH)
  b_emb2 = b_emb.reshape(1, DH)
  b_pred2 = b_pred.reshape(1, 1)

  deg2 = _deg_sc(edge_index, tail_dst)
  tab1, dis = _tc0(x, deg2, W_lin, b_lin2, W1)
  acc1 = _conv_sc(tab1, edge_index, tail_src, tail_dst)
  tab2, = _tc_mid(acc1, dis, b1_2, W2)
  acc2 = _conv_sc(tab2, edge_index, tail_src, tail_dst)
  tab3, = _tc_mid(acc2, dis, b2_2, W3)
  acc3 = _conv_sc(tab3, edge_index, tail_src, tail_dst)
  tabe, = _tc_emb(acc3, dis, b3_2, W_emb, b_emb2)
  accp = _pool_sc(tabe, src_p, dst_p)
  out, = _tc_pred(accp, W_pred, b_pred2)
  return out


# speedup vs baseline: 1.4474x; 1.4474x over previous
import functools

import jax
import jax.numpy as jnp
from jax import lax
from jax.experimental import pallas as pl
from jax.experimental.pallas import tpu as pltpu
from jax.experimental.pallas import tpu_sc as plsc

N = 50000
E = 800000
D_IN = 128
DH = 64
NG = 256
HALF = DH // 2

NP = 50176
NDUM = NP - N
NPOOL = 1280

C = 448
KM = 108
E_MAIN = 16 * KM * C
KT = 12
TAIL = 16 * KT * C
KMD = KM // 2
KTD = KT // 2
KP = 8
EPP = 16 * KP * C

BN = 1024


def _make_edge_scatter(num_rows, mode):
  mesh = plsc.VectorSubcoreMesh(core_axis_name="c", subcore_axis_name="s")
  zr = num_rows // 16
  gather = mode != "deg"

  scratch = []
  if gather:
    scratch += [pltpu.VMEM((C,), jnp.int32)] * 2
  scratch += [pltpu.VMEM((C,), jnp.int32)] * 2
  scratch += [pltpu.VMEM((C, HALF), jnp.float32)] * (2 if gather else 1)
  scratch += [
      pltpu.VMEM_SHARED((num_rows, HALF), jnp.float32),
  ]
  scratch += [pltpu.SemaphoreType.DMA] * 4

  def body(*refs):
    if mode == "conv":
      (table, edge, tsrc, tdst, out, sixa, sixb, dixa, dixb, rwa, rwb, acc,
       sga, sgb, ssa, ssb) = refs
    elif mode == "pool":
      (table, src1, dst1, out, sixa, sixb, dixa, dixb, rwa, rwb, acc,
       sga, sgb, ssa, ssb) = refs
    else:
      edge, tdst, out, dixa, dixb, rwa, acc, sga, sgb, ssa, ssb = refs
      rwb = rwa
    c = lax.axis_index("c")
    s = lax.axis_index("s")

    def fill_rows(val):
      v16 = jnp.full((16,), val, jnp.float32)
      def fb(r, carry):
        for j in range(HALF // 16):
          rwa[r, pl.ds(j * 16, 16)] = v16
        return carry
      lax.fori_loop(0, C, fb, 0)

    fill_rows(0.0)
    row0 = s * zr
    off = 0
    while off < zr:
      step = min(C, zr - off)
      pltpu.sync_copy(rwa.at[pl.ds(0, step)], acc.at[pl.ds(row0 + off, step)])
      off += step
    plsc.subcore_barrier()

    if not gather:
      fill_rows(1.0)

    if gather:
      tbl = table.at[c]

    def gat(six, rw, sem):
      return pltpu.make_async_copy(tbl.at[six], rw, sem)

    def sca(rw, dix, sem):
      return pltpu.make_async_copy(rw, acc.at[dix], sem)

    if mode == "deg":
      w = c * 16 + s
    else:
      w = s

    def load_main(k, six, dix):
      base = w * ((KMD if mode == "deg" else KM) * C) + k * C
      pltpu.sync_copy(edge.at[1, pl.ds(base, C)], dix)
      if gather:
        pltpu.sync_copy(edge.at[0, pl.ds(base, C)], six)

    def load_tail(k, six, dix):
      base = w * ((KTD if mode == "deg" else KT) * C) + k * C
      pltpu.sync_copy(tdst.at[pl.ds(base, C)], dix)
      if gather:
        pltpu.sync_copy(tsrc.at[pl.ds(base, C)], six)

    def load_pool(k, six, dix):
      base = s * (KP * C) + k * C
      pltpu.sync_copy(dst1.at[pl.ds(base, C)], dix)
      pltpu.sync_copy(src1.at[pl.ds(base, C)], six)

    def pipe(k_chunks, load):
      load(0, sixa, dixa)
      gat(sixa, rwa, sga).start()

      def pair(m, carry):
        load(2 * m + 1, sixb, dixb)
        gat(sixb, rwb, sgb).start()
        gat(sixa, rwa, sga).wait()
        sca(rwa, dixa, ssa).start(add=True)
        gat(sixb, rwb, sgb).wait()
        sca(rwb, dixb, ssb).start(add=True)
        sca(rwa, dixa, ssa).wait()
        load(2 * m + 2, sixa, dixa)
        gat(sixa, rwa, sga).start()
        sca(rwb, dixb, ssb).wait()
        return carry

      lax.fori_loop(0, k_chunks // 2 - 1, pair, 0)
      load(k_chunks - 1, sixb, dixb)
      gat(sixb, rwb, sgb).start()
      gat(sixa, rwa, sga).wait()
      sca(rwa, dixa, ssa).start(add=True)
      gat(sixb, rwb, sgb).wait()
      sca(rwb, dixb, ssb).start(add=True)
      sca(rwa, dixa, ssa).wait()
      sca(rwb, dixb, ssb).wait()

    def pipe_scatter_only(k_chunks, load):
      load(0, None, dixa)

      def pair(m, carry):
        sca(rwa, dixa, ssa).start(add=True)
        load(2 * m + 1, None, dixb)
        sca(rwb, dixb, ssb).start(add=True)
        sca(rwa, dixa, ssa).wait()
        load(2 * m + 2, None, dixa)
        sca(rwb, dixb, ssb).wait()
        return carry

      lax.fori_loop(0, k_chunks // 2 - 1, pair, 0)
      sca(rwa, dixa, ssa).start(add=True)
      load(k_chunks - 1, None, dixb)
      sca(rwb, dixb, ssb).start(add=True)
      sca(rwa, dixa, ssa).wait()
      sca(rwb, dixb, ssb).wait()

    if mode == "conv":
      pipe(KM, load_main)
      pipe(KT, load_tail)
    elif mode == "pool":
      pipe(KP, load_pool)
    else:
      pipe_scatter_only(KMD, load_main)
      pipe_scatter_only(KTD, load_tail)

    plsc.subcore_barrier()

    off = 0
    while off < zr:
      step = min(C, zr - off)
      pltpu.sync_copy(acc.at[pl.ds(row0 + off, step)], rwa.at[pl.ds(0, step)])
      pltpu.sync_copy(rwa.at[pl.ds(0, step)],
                      out.at[c, pl.ds(row0 + off, step)])
      off += step

  return pl.kernel(
      body,
      out_type=jax.ShapeDtypeStruct((2, num_rows, HALF), jnp.float32),
      mesh=mesh,
      scratch_types=scratch,
      compiler_params=pltpu.CompilerParams(use_tc_tiling_on_sc=False),
  )


_deg_sc = _make_edge_scatter(NP, "deg")
_conv_sc = _make_edge_scatter(NP, "conv")
_pool_sc = _make_edge_scatter(NPOOL, "pool")


_PACK = 128 // HALF
_BNP = BN // _PACK


def _tc0_body(xp_ref, deg_ref, wa_ref, wb_ref, ba_ref, bb_ref,
              tab_ref, dis_ref):
  dis = lax.rsqrt(deg_ref[0] + deg_ref[1])
  dis_ref[...] = dis
  xp = xp_ref[...]
  ta = jnp.dot(xp, wa_ref[...], preferred_element_type=jnp.float32)
  tb = jnp.dot(xp, wb_ref[...], preferred_element_type=jnp.float32)
  tab_ref[0] = (ta + ba_ref[...]) * dis
  tab_ref[1] = (tb + bb_ref[...]) * dis


def _tc_mid_body(acc_ref, dis_ref, waa_ref, wba_ref, wab_ref, wbb_ref,
                 ba_ref, bb_ref, tab_ref):
  dis = dis_ref[...]
  ca = jnp.maximum(dis * acc_ref[0] + ba_ref[...], 0.0)
  cb = jnp.maximum(dis * acc_ref[1] + bb_ref[...], 0.0)
  ta = (jnp.dot(ca, waa_ref[...], preferred_element_type=jnp.float32)
        + jnp.dot(cb, wba_ref[...], preferred_element_type=jnp.float32))
  tb = (jnp.dot(ca, wab_ref[...], preferred_element_type=jnp.float32)
        + jnp.dot(cb, wbb_ref[...], preferred_element_type=jnp.float32))
  tab_ref[0] = ta * dis
  tab_ref[1] = tb * dis


def _tc_emb_body(acc_ref, dis_ref, waa_ref, wba_ref, wab_ref, wbb_ref,
                 ba_ref, bb_ref, bea_ref, beb_ref, tab_ref):
  dis = dis_ref[...]
  ca = jnp.maximum(dis * acc_ref[0] + ba_ref[...], 0.0)
  cb = jnp.maximum(dis * acc_ref[1] + bb_ref[...], 0.0)
  ta = (jnp.dot(ca, waa_ref[...], preferred_element_type=jnp.float32)
        + jnp.dot(cb, wba_ref[...], preferred_element_type=jnp.float32))
  tb = (jnp.dot(ca, wab_ref[...], preferred_element_type=jnp.float32)
        + jnp.dot(cb, wbb_ref[...], preferred_element_type=jnp.float32))
  tab_ref[0] = jnp.maximum(ta + bea_ref[...], 0.0)
  tab_ref[1] = jnp.maximum(tb + beb_ref[...], 0.0)


def _tc_pred_body(accp_ref, wpred_ref, bpred_ref, out_ref):
  g = jnp.concatenate([accp_ref[0], accp_ref[1]], axis=1)
  out_ref[...] = (
      jnp.dot(g, wpred_ref[...], preferred_element_type=jnp.float32)
      + bpred_ref[...]
  )


def _full(shape):
  return pl.BlockSpec(shape, lambda i: tuple(0 for _ in shape))

_GRID = (N + BN - 1) // BN
_NPR = NP * HALF // 128
_acc_spec = pl.BlockSpec((2, _BNP, 128), lambda i: (0, i, 0))
_dis_spec = pl.BlockSpec((_BNP, 128), lambda i: (i, 0))
_tab_spec = pl.BlockSpec((2, _BNP, 128), lambda i: (0, i, 0))

_tab_shape = jax.ShapeDtypeStruct((2, _NPR, 128), jnp.float32)
_dis_shape = jax.ShapeDtypeStruct((_NPR, 128), jnp.float32)
_b128 = _full((1, 128))

_tc0 = pl.pallas_call(
    _tc0_body,
    grid=(_GRID,),
    in_specs=[
        pl.BlockSpec((_BNP, D_IN * _PACK), lambda i: (i, 0)),
        _acc_spec,
        _full((D_IN * _PACK, 128)),
        _full((D_IN * _PACK, 128)),
        _b128, _b128,
    ],
    out_specs=[_tab_spec, _dis_spec],
    out_shape=[_tab_shape, _dis_shape],
)

_w128 = _full((128, 128))

_tc_mid = pl.pallas_call(
    _tc_mid_body,
    grid=(_GRID,),
    in_specs=[_acc_spec, _dis_spec, _w128, _w128, _w128, _w128,
              _b128, _b128],
    out_specs=[_tab_spec],
    out_shape=[_tab_shape],
)

_tc_emb = pl.pallas_call(
    _tc_emb_body,
    grid=(_GRID,),
    in_specs=[_acc_spec, _dis_spec, _w128, _w128, _w128, _w128,
              _b128, _b128, _b128, _b128],
    out_specs=[_tab_spec],
    out_shape=[_tab_shape],
)

_tc_pred = pl.pallas_call(
    _tc_pred_body,
    grid=(1,),
    in_specs=[
        pl.BlockSpec((2, NG, HALF), lambda i: (0, 0, 0)),
        _full((DH, 1)),
        _full((1, 1)),
    ],
    out_specs=[pl.BlockSpec((NG, 1), lambda i: (0, 0))],
    out_shape=[jax.ShapeDtypeStruct((NG, 1), jnp.float32)],
)


def kernel(x, edge_index, batch, W_lin, b_lin, W1, b1, W2, b2, W3, b3,
           W_emb, b_emb, W_pred, b_pred):
  i32 = jnp.int32
  iota_n = jnp.arange(N, dtype=i32)

  npad = TAIL - (E - E_MAIN) - N
  pad = jnp.arange(npad, dtype=i32)
  tail_src = jnp.concatenate([edge_index[0, E_MAIN:], iota_n, pad % N])
  tail_dst = jnp.concatenate([edge_index[1, E_MAIN:], iota_n,
                              N + pad % NDUM])

  ppad = jnp.arange(EPP - N, dtype=i32)
  src_p = jnp.concatenate([iota_n, ppad % N])
  dst_p = jnp.concatenate([batch, NG + ppad % (NPOOL - NG)])

  f32 = jnp.float32
  eye = jnp.eye(_PACK, dtype=f32)

  def bd(m):
    return jnp.kron(eye, m)

  def bt(v):
    return jnp.tile(v, _PACK).reshape(1, 128)

  Wc = W_lin @ W1
  bc = b_lin @ W1
  b_pred2 = b_pred.reshape(1, 1)

  xp = x.reshape(N // _PACK, D_IN * _PACK)

  def packed(a):
    return a.reshape(2, _NPR, 128)

  def unpacked(t):
    return t.reshape(2, NP, HALF)

  deg2 = _deg_sc(edge_index, tail_dst)
  tab1, dis_p = _tc0(xp, packed(deg2),
                     bd(Wc[:, :HALF]), bd(Wc[:, HALF:]),
                     bt(bc[:HALF]), bt(bc[HALF:]))

  def mid(acc, b, W):
    t, = _tc_mid(packed(acc), dis_p,
                 bd(W[:HALF, :HALF]), bd(W[HALF:, :HALF]),
                 bd(W[:HALF, HALF:]), bd(W[HALF:, HALF:]),
                 bt(b[:HALF]), bt(b[HALF:]))
    return t

  acc1 = _conv_sc(unpacked(tab1), edge_index, tail_src, tail_dst)
  tab2 = mid(acc1, b1, W2)
  acc2 = _conv_sc(unpacked(tab2), edge_index, tail_src, tail_dst)
  tab3 = mid(acc2, b2, W3)
  acc3 = _conv_sc(unpacked(tab3), edge_index, tail_src, tail_dst)
  tabe, = _tc_emb(packed(acc3), dis_p,
                  bd(W_emb[:HALF, :HALF]), bd(W_emb[HALF:, :HALF]),
                  bd(W_emb[:HALF, HALF:]), bd(W_emb[HALF:, HALF:]),
                  bt(b3[:HALF]), bt(b3[HALF:]),
                  bt(b_emb[:HALF]), bt(b_emb[HALF:]))
  accp = _pool_sc(unpacked(tabe), src_p, dst_p)
  out, = _tc_pred(accp, W_pred, b_pred2)
  return out
